# baseline (speedup 1.0000x reference)
import functools

import jax
import jax.numpy as jnp
from jax import lax
from jax.experimental import pallas as pl
from jax.experimental.pallas import tpu as pltpu
from jax.experimental.pallas import tpu_sc as plsc

E = 8
TOP_K = 2
D_MODEL = 2048
D_FF = 1408
T = 2048

B = 512
S = T * TOP_K + E * B
NB = S // B

NC = 2
NS = 16
NW = NC * NS
TPW = T // NW
CH = 16


def _lane_cumsum(y, n):
    s = 1
    while s < n:
        z = jnp.zeros(y.shape[:1] + (s,), y.dtype)
        y = y + jnp.concatenate([z, y[:, :-s]], axis=1)
        s *= 2
    return y


def _route_body(rl_ref, pos_ref, wts_ref, meta_ref):
    r = rl_ref[...].astype(jnp.float32)
    row = lax.broadcasted_iota(jnp.int32, (E, T), 0)
    m1 = jnp.max(r, axis=0, keepdims=True)
    i1 = jnp.min(jnp.where(r == m1, row, E), axis=0, keepdims=True)
    mask1 = row == i1
    r2 = jnp.where(mask1, -jnp.inf, r)
    m2 = jnp.max(r2, axis=0, keepdims=True)
    i2 = jnp.min(jnp.where(r2 == m2, row, E), axis=0, keepdims=True)
    mask2 = row == i2

    p2 = jnp.exp(m2 - m1)
    z = 1.0 + p2
    w0 = 1.0 / z
    w1 = p2 / z
    pad_w = jnp.zeros((E - TOP_K, T), jnp.float32)
    wts_ref[...] = jnp.concatenate([w0, w1, pad_w], axis=0)

    oh0 = mask1.astype(jnp.int32)
    oh1 = mask2.astype(jnp.int32)
    ohs = oh0 + oh1
    cum = _lane_cumsum(ohs, T)
    count = lax.slice(cum, (0, T - 1), (E, T))
    pc = (count + B - 1) // B * B
    y = pc
    s = 1
    while s < E:
        z8 = jnp.zeros((s, 1), jnp.int32)
        y = y + jnp.concatenate([z8, y[:-s]], axis=0)
        s *= 2
    po = y - pc

    excl = cum - ohs
    r0 = jnp.sum(oh0 * (excl + po), axis=0, keepdims=True)
    r1 = jnp.sum(oh1 * (excl + po), axis=0, keepdims=True)
    pad_i = jnp.zeros((E - TOP_K, T), jnp.int32)
    pos_ref[...] = jnp.concatenate([r0, r1, pad_i], axis=0)

    lane = lax.broadcasted_iota(jnp.int32, (1, 128), 1)
    thr = po + pc
    be = jnp.sum((lane * B >= thr).astype(jnp.int32), axis=0, keepdims=True)
    be = jnp.clip(be, 0, E - 1)
    n_active = jnp.sum(pc) // B
    meta_row = jnp.where(lane == NB, n_active, be)
    meta_ref[...] = jnp.concatenate(
        [meta_row, jnp.zeros((7, 128), jnp.int32)], axis=0)


def _routing(router_logits):
    return pl.pallas_call(
        _route_body,
        out_shape=(
            jax.ShapeDtypeStruct((E, T), jnp.int32),
            jax.ShapeDtypeStruct((E, T), jnp.float32),
            jax.ShapeDtypeStruct((E, 128), jnp.int32),
        ),
    )(router_logits.T)


@functools.cache
def _sc_mesh():
    return plsc.VectorSubcoreMesh(
        core_axis_name="c", subcore_axis_name="s",
        num_cores=NC, num_subcores=NS)


@functools.cache
def _scatter_kernel():
    return pl.kernel(
        _scatter_x_body,
        out_type=jax.ShapeDtypeStruct((S, D_MODEL), jnp.float32),
        mesh=_sc_mesh(),
        scratch_types=[
            pltpu.VMEM((CH, D_MODEL), jnp.float32),
            pltpu.VMEM((CH,), jnp.int32),
            pltpu.VMEM((CH,), jnp.int32),
            pltpu.SemaphoreType.DMA,
        ],
    )


def _scatter_x_body(x_hbm, pos_hbm, xs_hbm, xrows_v, idx0_v, idx1_v, sem):
    wid = lax.axis_index("s") * NC + lax.axis_index("c")
    base = wid * TPW
    for c in range(TPW // CH):
        tb = base + c * CH
        pltpu.sync_copy(x_hbm.at[pl.ds(tb, CH)], xrows_v)
        pltpu.sync_copy(pos_hbm.at[0, pl.ds(tb, CH)], idx0_v)
        pltpu.sync_copy(pos_hbm.at[1, pl.ds(tb, CH)], idx1_v)
        pltpu.async_copy(xrows_v, xs_hbm.at[idx0_v], sem).wait()
        pltpu.async_copy(xrows_v, xs_hbm.at[idx1_v], sem).wait()


MD = 2
MB = D_MODEL // MD


def _gmm_body(meta_ref, xs_ref, w13_ref, w2_ref, ys_ref, h_scr):
    b = pl.program_id(0)
    md = pl.program_id(1)
    active = b < meta_ref[NB]
    dn = (((1,), (1,)), ((), ()))

    @pl.when(active & (md == 0))
    def _():
        xb = xs_ref[...].astype(jnp.bfloat16)
        w1 = w13_ref[0, :D_FF, :]
        w3 = w13_ref[0, D_FF:, :]
        g = lax.dot_general(xb, w1, dn, preferred_element_type=jnp.float32)
        u = lax.dot_general(xb, w3, dn, preferred_element_type=jnp.float32)
        h_scr[...] = (g * jax.nn.sigmoid(g) * u).astype(jnp.bfloat16)

    @pl.when(active)
    def _():
        w2b = w2_ref[0].astype(jnp.bfloat16)
        ys_ref[...] = lax.dot_general(
            h_scr[...], w2b, dn, preferred_element_type=jnp.float32)


def _grouped_matmul(meta, xs, w13_bf, w2):
    grid_spec = pltpu.PrefetchScalarGridSpec(
        num_scalar_prefetch=1,
        grid=(NB, MD),
        in_specs=[
            pl.BlockSpec((B, D_MODEL), lambda b, md, m: (b, 0)),
            pl.BlockSpec((1, 2 * D_FF, D_MODEL), lambda b, md, m: (m[b], 0, 0)),
            pl.BlockSpec((1, MB, D_FF), lambda b, md, m: (m[b], md, 0)),
        ],
        out_specs=pl.BlockSpec((B, MB), lambda b, md, m: (b, md)),
        scratch_shapes=[pltpu.VMEM((B, D_FF), jnp.bfloat16)],
    )
    return pl.pallas_call(
        _gmm_body,
        grid_spec=grid_spec,
        out_shape=jax.ShapeDtypeStruct((S, D_MODEL), jnp.float32),
    )(meta, xs, w13_bf, w2)


@functools.cache
def _gather2_kernel():
    return pl.kernel(
        _gather2_body,
        out_type=(
            jax.ShapeDtypeStruct((T, D_MODEL), jnp.float32),
            jax.ShapeDtypeStruct((T, D_MODEL), jnp.float32),
        ),
        mesh=_sc_mesh(),
        scratch_types=[
            pltpu.VMEM((CH, D_MODEL), jnp.float32),
            pltpu.VMEM((CH, D_MODEL), jnp.float32),
            pltpu.VMEM((CH,), jnp.int32),
            pltpu.VMEM((CH,), jnp.int32),
            pltpu.SemaphoreType.DMA,
        ],
    )


def _gather2_body(ys_hbm, pos_hbm, o0_hbm, o1_hbm,
                  r0_v, r1_v, idx0_v, idx1_v, sem):
    wid = lax.axis_index("s") * NC + lax.axis_index("c")
    base = wid * TPW
    for c in range(TPW // CH):
        tb = base + c * CH
        pltpu.sync_copy(pos_hbm.at[0, pl.ds(tb, CH)], idx0_v)
        pltpu.sync_copy(pos_hbm.at[1, pl.ds(tb, CH)], idx1_v)
        pltpu.async_copy(ys_hbm.at[idx0_v], r0_v, sem).wait()
        pltpu.async_copy(ys_hbm.at[idx1_v], r1_v, sem).wait()
        pltpu.sync_copy(r0_v, o0_hbm.at[pl.ds(tb, CH)])
        pltpu.sync_copy(r1_v, o1_hbm.at[pl.ds(tb, CH)])


BT2 = 512


def _wsum_body(o0_ref, o1_ref, wtc_ref, out_ref):
    w0 = wtc_ref[...][:, 0:1]
    w1 = wtc_ref[...][:, 1:2]
    out_ref[...] = w0 * o0_ref[...] + w1 * o1_ref[...]


def _weighted_sum(o0, o1, wtc):
    return pl.pallas_call(
        _wsum_body,
        grid=(T // BT2,),
        in_specs=[
            pl.BlockSpec((BT2, D_MODEL), lambda t: (t, 0)),
            pl.BlockSpec((BT2, D_MODEL), lambda t: (t, 0)),
            pl.BlockSpec((BT2, TOP_K), lambda t: (t, 0)),
        ],
        out_specs=pl.BlockSpec((BT2, D_MODEL), lambda t: (t, 0)),
        out_shape=jax.ShapeDtypeStruct((T, D_MODEL), jnp.float32),
    )(o0, o1, wtc)


@jax.jit
def kernel(x, router_logits, w13_weight, w2_weight):
    w13_bf = w13_weight.astype(jnp.bfloat16)
    pos, wts, meta8 = _routing(router_logits)
    meta = meta8[0]
    xs = _scatter_kernel()(x, pos)
    ys = _grouped_matmul(meta, xs, w13_bf, w2_weight)
    o0, o1 = _gather2_kernel()(ys, pos)
    wtc = wts[:TOP_K].T
    return _weighted_sum(o0, o1, wtc)


# speedup vs baseline: 1.1095x; 1.1095x over previous
import functools

import jax
import jax.numpy as jnp
from jax import lax
from jax.experimental import pallas as pl
from jax.experimental.pallas import tpu as pltpu
from jax.experimental.pallas import tpu_sc as plsc

E = 8
TOP_K = 2
D_MODEL = 2048
D_FF = 1408
T = 2048

B = 512
S = T * TOP_K + E * B
NB = S // B

NC = 2
NS = 16
NW = NC * NS
TPW = T // NW
CH = 16


def _lane_cumsum(y, n):
    s = 1
    while s < n:
        z = jnp.zeros(y.shape[:1] + (s,), y.dtype)
        y = y + jnp.concatenate([z, y[:, :-s]], axis=1)
        s *= 2
    return y


def _route_body(rl_ref, pos_ref, wts_ref, meta_ref):
    r = rl_ref[...].astype(jnp.float32)
    row = lax.broadcasted_iota(jnp.int32, (E, T), 0)
    m1 = jnp.max(r, axis=0, keepdims=True)
    i1 = jnp.min(jnp.where(r == m1, row, E), axis=0, keepdims=True)
    mask1 = row == i1
    r2 = jnp.where(mask1, -jnp.inf, r)
    m2 = jnp.max(r2, axis=0, keepdims=True)
    i2 = jnp.min(jnp.where(r2 == m2, row, E), axis=0, keepdims=True)
    mask2 = row == i2

    p2 = jnp.exp(m2 - m1)
    z = 1.0 + p2
    w0 = 1.0 / z
    w1 = p2 / z
    pad_w = jnp.zeros((E - TOP_K, T), jnp.float32)
    wts_ref[...] = jnp.concatenate([w0, w1, pad_w], axis=0)

    oh0 = mask1.astype(jnp.int32)
    oh1 = mask2.astype(jnp.int32)
    ohs = oh0 + oh1
    cum = _lane_cumsum(ohs, T)
    count = lax.slice(cum, (0, T - 1), (E, T))
    pc = (count + B - 1) // B * B
    y = pc
    s = 1
    while s < E:
        z8 = jnp.zeros((s, 1), jnp.int32)
        y = y + jnp.concatenate([z8, y[:-s]], axis=0)
        s *= 2
    po = y - pc

    excl = cum - ohs
    r0 = jnp.sum(oh0 * (excl + po), axis=0, keepdims=True)
    r1 = jnp.sum(oh1 * (excl + po), axis=0, keepdims=True)
    pad_i = jnp.zeros((E - TOP_K, T), jnp.int32)
    pos_ref[...] = jnp.concatenate([r0, r1, pad_i], axis=0)

    lane = lax.broadcasted_iota(jnp.int32, (1, 128), 1)
    thr = po + pc
    be = jnp.sum((lane * B >= thr).astype(jnp.int32), axis=0, keepdims=True)
    be = jnp.clip(be, 0, E - 1)
    n_active = jnp.sum(pc) // B
    meta_row = jnp.where(lane == NB, n_active, be)
    meta_ref[...] = jnp.concatenate(
        [meta_row, jnp.zeros((7, 128), jnp.int32)], axis=0)


def _routing(router_logits):
    return pl.pallas_call(
        _route_body,
        out_shape=(
            jax.ShapeDtypeStruct((E, T), jnp.int32),
            jax.ShapeDtypeStruct((E, T), jnp.float32),
            jax.ShapeDtypeStruct((E, 128), jnp.int32),
        ),
    )(router_logits.T)


@functools.cache
def _sc_mesh():
    return plsc.VectorSubcoreMesh(
        core_axis_name="c", subcore_axis_name="s",
        num_cores=NC, num_subcores=NS)


@functools.cache
def _scatter_kernel():
    return pl.kernel(
        _scatter_x_body,
        out_type=jax.ShapeDtypeStruct((S, D_MODEL), jnp.float32),
        mesh=_sc_mesh(),
        scratch_types=[
            pltpu.VMEM((CH, D_MODEL), jnp.float32),
            pltpu.VMEM((CH,), jnp.int32),
            pltpu.VMEM((CH,), jnp.int32),
            pltpu.SemaphoreType.DMA,
        ],
    )


def _scatter_x_body(x_hbm, pos_hbm, xs_hbm, xrows_v, idx0_v, idx1_v, sem):
    wid = lax.axis_index("s") * NC + lax.axis_index("c")
    base = wid * TPW
    for c in range(TPW // CH):
        tb = base + c * CH
        pltpu.sync_copy(x_hbm.at[pl.ds(tb, CH)], xrows_v)
        pltpu.sync_copy(pos_hbm.at[0, pl.ds(tb, CH)], idx0_v)
        pltpu.sync_copy(pos_hbm.at[1, pl.ds(tb, CH)], idx1_v)
        pltpu.async_copy(xrows_v, xs_hbm.at[idx0_v], sem).wait()
        pltpu.async_copy(xrows_v, xs_hbm.at[idx1_v], sem).wait()


def _gmm_body(meta_ref, xs_ref, w13_ref, w2_ref, ys_ref):
    b = pl.program_id(0)

    @pl.when(b < meta_ref[NB])
    def _():
        xb = xs_ref[...].astype(jnp.bfloat16)
        w1 = w13_ref[0, :D_FF, :]
        w3 = w13_ref[0, D_FF:, :]
        dn = (((1,), (1,)), ((), ()))
        g = lax.dot_general(xb, w1, dn, preferred_element_type=jnp.float32)
        u = lax.dot_general(xb, w3, dn, preferred_element_type=jnp.float32)
        h = (g * jax.nn.sigmoid(g) * u).astype(jnp.bfloat16)
        ys_ref[...] = lax.dot_general(
            h, w2_ref[0], dn, preferred_element_type=jnp.float32)


def _grouped_matmul(meta, xs, w13_bf, w2_bf):
    grid_spec = pltpu.PrefetchScalarGridSpec(
        num_scalar_prefetch=1,
        grid=(NB,),
        in_specs=[
            pl.BlockSpec((B, D_MODEL), lambda b, m: (b, 0)),
            pl.BlockSpec((1, 2 * D_FF, D_MODEL), lambda b, m: (m[b], 0, 0)),
            pl.BlockSpec((1, D_MODEL, D_FF), lambda b, m: (m[b], 0, 0)),
        ],
        out_specs=pl.BlockSpec((B, D_MODEL), lambda b, m: (b, 0)),
    )
    return pl.pallas_call(
        _gmm_body,
        grid_spec=grid_spec,
        out_shape=jax.ShapeDtypeStruct((S, D_MODEL), jnp.float32),
    )(meta, xs, w13_bf, w2_bf)


@functools.cache
def _gather2_kernel():
    return pl.kernel(
        _gather2_body,
        out_type=(
            jax.ShapeDtypeStruct((T, D_MODEL), jnp.float32),
            jax.ShapeDtypeStruct((T, D_MODEL), jnp.float32),
        ),
        mesh=_sc_mesh(),
        scratch_types=[
            pltpu.VMEM((CH, D_MODEL), jnp.float32),
            pltpu.VMEM((CH, D_MODEL), jnp.float32),
            pltpu.VMEM((CH,), jnp.int32),
            pltpu.VMEM((CH,), jnp.int32),
            pltpu.SemaphoreType.DMA,
        ],
    )


def _gather2_body(ys_hbm, pos_hbm, o0_hbm, o1_hbm,
                  r0_v, r1_v, idx0_v, idx1_v, sem):
    wid = lax.axis_index("s") * NC + lax.axis_index("c")
    base = wid * TPW
    for c in range(TPW // CH):
        tb = base + c * CH
        pltpu.sync_copy(pos_hbm.at[0, pl.ds(tb, CH)], idx0_v)
        pltpu.sync_copy(pos_hbm.at[1, pl.ds(tb, CH)], idx1_v)
        pltpu.async_copy(ys_hbm.at[idx0_v], r0_v, sem).wait()
        pltpu.async_copy(ys_hbm.at[idx1_v], r1_v, sem).wait()
        pltpu.sync_copy(r0_v, o0_hbm.at[pl.ds(tb, CH)])
        pltpu.sync_copy(r1_v, o1_hbm.at[pl.ds(tb, CH)])


BT2 = 512


def _wsum_body(o0_ref, o1_ref, wtc_ref, out_ref):
    w0 = wtc_ref[...][:, 0:1]
    w1 = wtc_ref[...][:, 1:2]
    out_ref[...] = w0 * o0_ref[...] + w1 * o1_ref[...]


def _weighted_sum(o0, o1, wtc):
    return pl.pallas_call(
        _wsum_body,
        grid=(T // BT2,),
        in_specs=[
            pl.BlockSpec((BT2, D_MODEL), lambda t: (t, 0)),
            pl.BlockSpec((BT2, D_MODEL), lambda t: (t, 0)),
            pl.BlockSpec((BT2, TOP_K), lambda t: (t, 0)),
        ],
        out_specs=pl.BlockSpec((BT2, D_MODEL), lambda t: (t, 0)),
        out_shape=jax.ShapeDtypeStruct((T, D_MODEL), jnp.float32),
    )(o0, o1, wtc)


@jax.jit
def kernel(x, router_logits, w13_weight, w2_weight):
    w13_bf = w13_weight.astype(jnp.bfloat16)
    w2_bf = w2_weight.astype(jnp.bfloat16)
    pos, wts, meta8 = _routing(router_logits)
    meta = meta8[0]
    xs = _scatter_kernel()(x, pos)
    ys = _grouped_matmul(meta, xs, w13_bf, w2_bf)
    o0, o1 = _gather2_kernel()(ys, pos)
    wtc = wts[:TOP_K].T
    return _weighted_sum(o0, o1, wtc)


# speedup vs baseline: 1.1452x; 1.0322x over previous
import functools

import jax
import jax.numpy as jnp
from jax import lax
from jax.experimental import pallas as pl
from jax.experimental.pallas import tpu as pltpu
from jax.experimental.pallas import tpu_sc as plsc

E = 8
TOP_K = 2
D_MODEL = 2048
D_FF = 1408
T = 2048

B = 512
S = T * TOP_K + E * B
NB = S // B

NC = 2
NS = 16
NW = NC * NS
TPW = T // NW
CH = 16


def _lane_cumsum(y, n):
    s = 1
    while s < n:
        z = jnp.zeros(y.shape[:1] + (s,), y.dtype)
        y = y + jnp.concatenate([z, y[:, :-s]], axis=1)
        s *= 2
    return y


def _route_body(rl_ref, pos_ref, wts_ref, meta_ref):
    r = rl_ref[...].astype(jnp.float32)
    row = lax.broadcasted_iota(jnp.int32, (E, T), 0)
    m1 = jnp.max(r, axis=0, keepdims=True)
    i1 = jnp.min(jnp.where(r == m1, row, E), axis=0, keepdims=True)
    mask1 = row == i1
    r2 = jnp.where(mask1, -jnp.inf, r)
    m2 = jnp.max(r2, axis=0, keepdims=True)
    i2 = jnp.min(jnp.where(r2 == m2, row, E), axis=0, keepdims=True)
    mask2 = row == i2

    p2 = jnp.exp(m2 - m1)
    z = 1.0 + p2
    w0 = 1.0 / z
    w1 = p2 / z
    pad_w = jnp.zeros((E - TOP_K, T), jnp.float32)
    wts_ref[...] = jnp.concatenate([w0, w1, pad_w], axis=0)

    oh0 = mask1.astype(jnp.int32)
    oh1 = mask2.astype(jnp.int32)
    ohs = oh0 + oh1
    cum = _lane_cumsum(ohs, T)
    count = lax.slice(cum, (0, T - 1), (E, T))
    pc = (count + B - 1) // B * B
    y = pc
    s = 1
    while s < E:
        z8 = jnp.zeros((s, 1), jnp.int32)
        y = y + jnp.concatenate([z8, y[:-s]], axis=0)
        s *= 2
    po = y - pc

    excl = cum - ohs
    r0 = jnp.sum(oh0 * (excl + po), axis=0, keepdims=True)
    r1 = jnp.sum(oh1 * (excl + po), axis=0, keepdims=True)
    pad_i = jnp.zeros((E - TOP_K, T), jnp.int32)
    pos_ref[...] = jnp.concatenate([r0, r1, pad_i], axis=0)

    lane = lax.broadcasted_iota(jnp.int32, (1, 128), 1)
    thr = po + pc
    be = jnp.sum((lane * B >= thr).astype(jnp.int32), axis=0, keepdims=True)
    be = jnp.clip(be, 0, E - 1)
    n_active = jnp.sum(pc) // B
    meta_row = jnp.where(lane == NB, n_active, be)
    meta_ref[...] = jnp.concatenate(
        [meta_row, jnp.zeros((7, 128), jnp.int32)], axis=0)


def _routing(router_logits):
    return pl.pallas_call(
        _route_body,
        out_shape=(
            jax.ShapeDtypeStruct((E, T), jnp.int32),
            jax.ShapeDtypeStruct((E, T), jnp.float32),
            jax.ShapeDtypeStruct((E, 128), jnp.int32),
        ),
    )(router_logits.T)


@functools.cache
def _sc_mesh():
    return plsc.VectorSubcoreMesh(
        core_axis_name="c", subcore_axis_name="s",
        num_cores=NC, num_subcores=NS)


@functools.cache
def _scatter_kernel():
    return pl.kernel(
        _scatter_x_body,
        out_type=jax.ShapeDtypeStruct((S, D_MODEL), jnp.float32),
        mesh=_sc_mesh(),
        scratch_types=[
            pltpu.VMEM((CH, D_MODEL), jnp.float32),
            pltpu.VMEM((CH,), jnp.int32),
            pltpu.VMEM((CH,), jnp.int32),
            pltpu.SemaphoreType.DMA,
        ],
    )


def _scatter_x_body(x_hbm, pos_hbm, xs_hbm, xrows_v, idx0_v, idx1_v, sem):
    wid = lax.axis_index("s") * NC + lax.axis_index("c")
    base = wid * TPW
    for c in range(TPW // CH):
        tb = base + c * CH
        pltpu.sync_copy(x_hbm.at[pl.ds(tb, CH)], xrows_v)
        pltpu.sync_copy(pos_hbm.at[0, pl.ds(tb, CH)], idx0_v)
        pltpu.sync_copy(pos_hbm.at[1, pl.ds(tb, CH)], idx1_v)
        pltpu.async_copy(xrows_v, xs_hbm.at[idx0_v], sem).wait()
        pltpu.async_copy(xrows_v, xs_hbm.at[idx1_v], sem).wait()


def _gmm_body(meta_ref, xs_ref, w13_ref, w2_ref, ys_ref):
    b = pl.program_id(0)

    @pl.when(b < meta_ref[NB])
    def _():
        xb = xs_ref[...].astype(jnp.bfloat16)
        w1 = w13_ref[0, :D_FF, :]
        w3 = w13_ref[0, D_FF:, :]
        dn = (((1,), (1,)), ((), ()))
        g = lax.dot_general(xb, w1, dn, preferred_element_type=jnp.float32)
        u = lax.dot_general(xb, w3, dn, preferred_element_type=jnp.float32)
        h = (g * jax.nn.sigmoid(g) * u).astype(jnp.bfloat16)
        ys_ref[...] = lax.dot_general(
            h, w2_ref[0], dn, preferred_element_type=jnp.float32)


def _grouped_matmul(meta, xs, w13_bf, w2_bf):
    grid_spec = pltpu.PrefetchScalarGridSpec(
        num_scalar_prefetch=1,
        grid=(NB,),
        in_specs=[
            pl.BlockSpec((B, D_MODEL),
                         lambda b, m: (jnp.minimum(b, m[NB] - 1), 0)),
            pl.BlockSpec((1, 2 * D_FF, D_MODEL), lambda b, m: (m[b], 0, 0)),
            pl.BlockSpec((1, D_MODEL, D_FF), lambda b, m: (m[b], 0, 0)),
        ],
        out_specs=pl.BlockSpec((B, D_MODEL),
                               lambda b, m: (jnp.minimum(b, m[NB] - 1), 0)),
    )
    return pl.pallas_call(
        _gmm_body,
        grid_spec=grid_spec,
        out_shape=jax.ShapeDtypeStruct((S, D_MODEL), jnp.float32),
    )(meta, xs, w13_bf, w2_bf)


@functools.cache
def _gather2_kernel():
    return pl.kernel(
        _gather2_body,
        out_type=(
            jax.ShapeDtypeStruct((T, D_MODEL), jnp.float32),
            jax.ShapeDtypeStruct((T, D_MODEL), jnp.float32),
        ),
        mesh=_sc_mesh(),
        scratch_types=[
            pltpu.VMEM((CH, D_MODEL), jnp.float32),
            pltpu.VMEM((CH, D_MODEL), jnp.float32),
            pltpu.VMEM((CH,), jnp.int32),
            pltpu.VMEM((CH,), jnp.int32),
            pltpu.SemaphoreType.DMA,
            pltpu.SemaphoreType.DMA,
        ],
    )


def _gather2_body(ys_hbm, pos_hbm, o0_hbm, o1_hbm,
                  r0_v, r1_v, idx0_v, idx1_v, sem0, sem1):
    wid = lax.axis_index("s") * NC + lax.axis_index("c")
    base = wid * TPW
    for c in range(TPW // CH):
        tb = base + c * CH
        pltpu.sync_copy(pos_hbm.at[0, pl.ds(tb, CH)], idx0_v)
        pltpu.sync_copy(pos_hbm.at[1, pl.ds(tb, CH)], idx1_v)
        g0 = pltpu.async_copy(ys_hbm.at[idx0_v], r0_v, sem0)
        g1 = pltpu.async_copy(ys_hbm.at[idx1_v], r1_v, sem1)
        g0.wait()
        g1.wait()
        pltpu.sync_copy(r0_v, o0_hbm.at[pl.ds(tb, CH)])
        pltpu.sync_copy(r1_v, o1_hbm.at[pl.ds(tb, CH)])


BT2 = 512


def _wsum_body(o0_ref, o1_ref, wtc_ref, out_ref):
    w0 = wtc_ref[...][:, 0:1]
    w1 = wtc_ref[...][:, 1:2]
    out_ref[...] = w0 * o0_ref[...] + w1 * o1_ref[...]


def _weighted_sum(o0, o1, wtc):
    return pl.pallas_call(
        _wsum_body,
        grid=(T // BT2,),
        in_specs=[
            pl.BlockSpec((BT2, D_MODEL), lambda t: (t, 0)),
            pl.BlockSpec((BT2, D_MODEL), lambda t: (t, 0)),
            pl.BlockSpec((BT2, TOP_K), lambda t: (t, 0)),
        ],
        out_specs=pl.BlockSpec((BT2, D_MODEL), lambda t: (t, 0)),
        out_shape=jax.ShapeDtypeStruct((T, D_MODEL), jnp.float32),
    )(o0, o1, wtc)


@jax.jit
def kernel(x, router_logits, w13_weight, w2_weight):
    w13_bf = w13_weight.astype(jnp.bfloat16)
    w2_bf = w2_weight.astype(jnp.bfloat16)
    pos, wts, meta8 = _routing(router_logits)
    meta = meta8[0]
    xs = _scatter_kernel()(x, pos)
    ys = _grouped_matmul(meta, xs, w13_bf, w2_bf)
    o0, o1 = _gather2_kernel()(ys, pos)
    wtc = wts[:TOP_K].T
    return _weighted_sum(o0, o1, wtc)


# speedup vs baseline: 1.2578x; 1.0983x over previous
import functools

import jax
import jax.numpy as jnp
from jax import lax
from jax.experimental import pallas as pl
from jax.experimental.pallas import tpu as pltpu
from jax.experimental.pallas import tpu_sc as plsc

E = 8
TOP_K = 2
D_MODEL = 2048
D_FF = 1408
T = 2048

B = 256
S = T * TOP_K + E * B
NB = S // B

NC = 2
NS = 16
NW = NC * NS
TPW = T // NW
CH = 16


def _lane_cumsum(y, n):
    s = 1
    while s < n:
        z = jnp.zeros(y.shape[:1] + (s,), y.dtype)
        y = y + jnp.concatenate([z, y[:, :-s]], axis=1)
        s *= 2
    return y


def _route_body(rl_ref, pos_ref, wts_ref, meta_ref):
    r = rl_ref[...].astype(jnp.float32)
    row = lax.broadcasted_iota(jnp.int32, (E, T), 0)
    m1 = jnp.max(r, axis=0, keepdims=True)
    i1 = jnp.min(jnp.where(r == m1, row, E), axis=0, keepdims=True)
    mask1 = row == i1
    r2 = jnp.where(mask1, -jnp.inf, r)
    m2 = jnp.max(r2, axis=0, keepdims=True)
    i2 = jnp.min(jnp.where(r2 == m2, row, E), axis=0, keepdims=True)
    mask2 = row == i2

    p2 = jnp.exp(m2 - m1)
    z = 1.0 + p2
    w0 = 1.0 / z
    w1 = p2 / z
    pad_w = jnp.zeros((E - TOP_K, T), jnp.float32)
    wts_ref[...] = jnp.concatenate([w0, w1, pad_w], axis=0)

    oh0 = mask1.astype(jnp.int32)
    oh1 = mask2.astype(jnp.int32)
    ohs = oh0 + oh1
    cum = _lane_cumsum(ohs, T)
    count = lax.slice(cum, (0, T - 1), (E, T))
    pc = (count + B - 1) // B * B
    y = pc
    s = 1
    while s < E:
        z8 = jnp.zeros((s, 1), jnp.int32)
        y = y + jnp.concatenate([z8, y[:-s]], axis=0)
        s *= 2
    po = y - pc

    excl = cum - ohs
    r0 = jnp.sum(oh0 * (excl + po), axis=0, keepdims=True)
    r1 = jnp.sum(oh1 * (excl + po), axis=0, keepdims=True)
    pad_i = jnp.zeros((E - TOP_K, T), jnp.int32)
    pos_ref[...] = jnp.concatenate([r0, r1, pad_i], axis=0)

    lane = lax.broadcasted_iota(jnp.int32, (1, 128), 1)
    thr = po + pc
    be = jnp.sum((lane * B >= thr).astype(jnp.int32), axis=0, keepdims=True)
    be = jnp.clip(be, 0, E - 1)
    n_active = jnp.sum(pc) // B
    meta_row = jnp.where(lane == NB, n_active, be)
    meta_ref[...] = jnp.concatenate(
        [meta_row, jnp.zeros((7, 128), jnp.int32)], axis=0)


def _routing(router_logits):
    return pl.pallas_call(
        _route_body,
        out_shape=(
            jax.ShapeDtypeStruct((E, T), jnp.int32),
            jax.ShapeDtypeStruct((E, T), jnp.float32),
            jax.ShapeDtypeStruct((E, 128), jnp.int32),
        ),
    )(router_logits.T)


@functools.cache
def _sc_mesh():
    return plsc.VectorSubcoreMesh(
        core_axis_name="c", subcore_axis_name="s",
        num_cores=NC, num_subcores=NS)


@functools.cache
def _scatter_kernel():
    return pl.kernel(
        _scatter_x_body,
        out_type=jax.ShapeDtypeStruct((S, D_MODEL), jnp.float32),
        mesh=_sc_mesh(),
        scratch_types=[
            pltpu.VMEM((CH, D_MODEL), jnp.float32),
            pltpu.VMEM((CH,), jnp.int32),
            pltpu.VMEM((CH,), jnp.int32),
            pltpu.SemaphoreType.DMA,
        ],
    )


def _scatter_x_body(x_hbm, pos_hbm, xs_hbm, xrows_v, idx0_v, idx1_v, sem):
    wid = lax.axis_index("s") * NC + lax.axis_index("c")
    base = wid * TPW
    for c in range(TPW // CH):
        tb = base + c * CH
        pltpu.sync_copy(x_hbm.at[pl.ds(tb, CH)], xrows_v)
        pltpu.sync_copy(pos_hbm.at[0, pl.ds(tb, CH)], idx0_v)
        pltpu.sync_copy(pos_hbm.at[1, pl.ds(tb, CH)], idx1_v)
        pltpu.async_copy(xrows_v, xs_hbm.at[idx0_v], sem).wait()
        pltpu.async_copy(xrows_v, xs_hbm.at[idx1_v], sem).wait()


def _gmm_body(meta_ref, xs_ref, w13_ref, w2_ref, ys_ref):
    b = pl.program_id(0)

    @pl.when(b < meta_ref[NB])
    def _():
        xb = xs_ref[...].astype(jnp.bfloat16)
        w1 = w13_ref[0, :D_FF, :]
        w3 = w13_ref[0, D_FF:, :]
        dn = (((1,), (1,)), ((), ()))
        g = lax.dot_general(xb, w1, dn, preferred_element_type=jnp.float32)
        u = lax.dot_general(xb, w3, dn, preferred_element_type=jnp.float32)
        h = (g * jax.nn.sigmoid(g) * u).astype(jnp.bfloat16)
        w2b = w2_ref[0].astype(jnp.bfloat16)
        ys_ref[...] = lax.dot_general(
            h, w2b, dn, preferred_element_type=jnp.float32)


def _grouped_matmul(meta, xs, w13_bf, w2_bf):
    grid_spec = pltpu.PrefetchScalarGridSpec(
        num_scalar_prefetch=1,
        grid=(NB,),
        in_specs=[
            pl.BlockSpec((B, D_MODEL),
                         lambda b, m: (jnp.minimum(b, m[NB] - 1), 0)),
            pl.BlockSpec((1, 2 * D_FF, D_MODEL), lambda b, m: (m[b], 0, 0)),
            pl.BlockSpec((1, D_MODEL, D_FF), lambda b, m: (m[b], 0, 0)),
        ],
        out_specs=pl.BlockSpec((B, D_MODEL),
                               lambda b, m: (jnp.minimum(b, m[NB] - 1), 0)),
    )
    return pl.pallas_call(
        _gmm_body,
        grid_spec=grid_spec,
        out_shape=jax.ShapeDtypeStruct((S, D_MODEL), jnp.float32),
    )(meta, xs, w13_bf, w2_bf)


@functools.cache
def _gather2_kernel():
    return pl.kernel(
        _gather2_body,
        out_type=(
            jax.ShapeDtypeStruct((T, D_MODEL), jnp.float32),
            jax.ShapeDtypeStruct((T, D_MODEL), jnp.float32),
        ),
        mesh=_sc_mesh(),
        scratch_types=[
            pltpu.VMEM((CH, D_MODEL), jnp.float32),
            pltpu.VMEM((CH, D_MODEL), jnp.float32),
            pltpu.VMEM((CH,), jnp.int32),
            pltpu.VMEM((CH,), jnp.int32),
            pltpu.SemaphoreType.DMA,
            pltpu.SemaphoreType.DMA,
        ],
    )


def _gather2_body(ys_hbm, pos_hbm, o0_hbm, o1_hbm,
                  r0_v, r1_v, idx0_v, idx1_v, sem0, sem1):
    wid = lax.axis_index("s") * NC + lax.axis_index("c")
    base = wid * TPW
    for c in range(TPW // CH):
        tb = base + c * CH
        pltpu.sync_copy(pos_hbm.at[0, pl.ds(tb, CH)], idx0_v)
        pltpu.sync_copy(pos_hbm.at[1, pl.ds(tb, CH)], idx1_v)
        g0 = pltpu.async_copy(ys_hbm.at[idx0_v], r0_v, sem0)
        g1 = pltpu.async_copy(ys_hbm.at[idx1_v], r1_v, sem1)
        g0.wait()
        g1.wait()
        pltpu.sync_copy(r0_v, o0_hbm.at[pl.ds(tb, CH)])
        pltpu.sync_copy(r1_v, o1_hbm.at[pl.ds(tb, CH)])


BT2 = 512


def _wsum_body(o0_ref, o1_ref, wtc_ref, out_ref):
    w0 = wtc_ref[...][:, 0:1]
    w1 = wtc_ref[...][:, 1:2]
    out_ref[...] = w0 * o0_ref[...] + w1 * o1_ref[...]


def _weighted_sum(o0, o1, wtc):
    return pl.pallas_call(
        _wsum_body,
        grid=(T // BT2,),
        in_specs=[
            pl.BlockSpec((BT2, D_MODEL), lambda t: (t, 0)),
            pl.BlockSpec((BT2, D_MODEL), lambda t: (t, 0)),
            pl.BlockSpec((BT2, TOP_K), lambda t: (t, 0)),
        ],
        out_specs=pl.BlockSpec((BT2, D_MODEL), lambda t: (t, 0)),
        out_shape=jax.ShapeDtypeStruct((T, D_MODEL), jnp.float32),
    )(o0, o1, wtc)


@jax.jit
def kernel(x, router_logits, w13_weight, w2_weight):
    w13_bf = w13_weight.astype(jnp.bfloat16)
    pos, wts, meta8 = _routing(router_logits)
    meta = meta8[0]
    xs = _scatter_kernel()(x, pos)
    ys = _grouped_matmul(meta, xs, w13_bf, w2_weight)
    o0, o1 = _gather2_kernel()(ys, pos)
    wtc = wts[:TOP_K].T
    return _weighted_sum(o0, o1, wtc)
